# Initial kernel scaffold; baseline (speedup 1.0000x reference)
import jax
import jax.numpy as jnp
from jax.experimental import pallas as pl


def kernel(x, w):
    raise NotImplementedError("write your pallas kernel here")


# speedup vs baseline: 254.9405x; 254.9405x over previous
import functools

import jax
import jax.numpy as jnp
from jax import lax
from jax.experimental import pallas as pl
from jax.experimental.pallas import tpu as pltpu
from jax.experimental.pallas import tpu_sc as plsc

B, O, I, D = 32, 16, 16, 1024
K = 1025
SEGMENTS = 512
OG = 4
BG = 4
NB_GROUPS = B // BG
NO_GROUPS = O // OG
TABLE_WORDS = OG * I * K
XSLAB = I * D
ACC_WORDS = OG * D
NV = D // 16


def _body(w_hbm, x_hbm, out_hbm, table_v, xb_v, acc_v):
    wid = lax.axis_index("s") * 2 + lax.axis_index("c")
    o_group = wid % NO_GROUPS
    b_group = wid // NO_GROUPS

    pltpu.sync_copy(w_hbm.at[pl.ds(o_group * TABLE_WORDS, TABLE_WORDS)], table_v)

    for b in range(BG):
        b_abs = b_group * BG + b
        pltpu.sync_copy(x_hbm.at[pl.ds(b_abs * XSLAB, XSLAB)], xb_v)

        def dloop(v, _):
            dv = v * 16
            acc = [jnp.zeros((16,), jnp.float32) for _ in range(OG)]
            for i in range(I):
                xv = xb_v[pl.ds(i * D + dv, 16)]
                seg = ((xv + 1.0) * 256.0).astype(jnp.int32)
                seg = jnp.minimum(jnp.maximum(seg, 0), SEGMENTS - 1)
                x_min = seg.astype(jnp.float32) * (1.0 / 256.0) - 1.0
                t = (xv - x_min) * 512.0 - 1.0
                t2 = t * t
                c0 = 0.5 * (t2 - t)
                c1 = 1.0 - t2
                c2 = 0.5 * (t2 + t)
                base = 2 * seg
                for o in range(OG):
                    ro = (o * I + i) * K
                    g0 = plsc.load_gather(table_v, [base + ro])
                    g1 = plsc.load_gather(table_v, [base + (ro + 1)])
                    g2 = plsc.load_gather(table_v, [base + (ro + 2)])
                    acc[o] = acc[o] + c0 * g0 + c1 * g1 + c2 * g2
            for o in range(OG):
                acc_v[pl.ds(o * D + dv, 16)] = acc[o]
            return ()

        lax.fori_loop(0, NV, dloop, ())
        out_off = (b_abs * O + o_group * OG) * D
        pltpu.sync_copy(acc_v, out_hbm.at[pl.ds(out_off, ACC_WORDS)])


@jax.jit
def _piecewise_sc(x_flat, w_flat):
    mesh = plsc.VectorSubcoreMesh(core_axis_name="c", subcore_axis_name="s")
    kfn = functools.partial(
        pl.kernel,
        mesh=mesh,
        out_type=jax.ShapeDtypeStruct((B * O * D,), jnp.float32),
        scratch_types=[
            pltpu.VMEM((TABLE_WORDS,), jnp.float32),
            pltpu.VMEM((XSLAB,), jnp.float32),
            pltpu.VMEM((ACC_WORDS,), jnp.float32),
        ],
        compiler_params=pltpu.CompilerParams(needs_layout_passes=False),
    )(_body)
    return kfn(w_flat, x_flat)


def kernel(x, w):
    x_flat = x.reshape(B * I * D)
    w_flat = w.reshape(O * I * K)
    out = _piecewise_sc(x_flat, w_flat)
    return out.reshape(B, O, D)
